# baseline (speedup 1.0000x reference)
import functools

import jax
import jax.numpy as jnp
from jax import lax
from jax.experimental import pallas as pl
from jax.experimental.pallas import tpu as pltpu
from jax.experimental.pallas import tpu_sc as plsc

N_GRAPH = 64
NW = 32
CHUNK = 3136
VECS = CHUNK // 16


def _sc_bincount_body(idx_hbm, out_hbm, chunk_ref, hist_ref, sem):
    wid = lax.axis_index("c") * 16 + lax.axis_index("s")
    base = wid * CHUNK
    pltpu.sync_copy(idx_hbm.at[pl.ds(base, CHUNK)], chunk_ref)

    lanes = jax.lax.iota(jnp.int32, 16)
    lo = chunk_ref[pl.ds(0, 16)][0]
    hi = chunk_ref[pl.ds(CHUNK - 16, 16)][15]

    def count_less(t):
        def step(_, lo_hi):
            lo_b, hi_b = lo_hi
            active = lo_b < hi_b
            mid = jnp.minimum((lo_b + hi_b) // 2, jnp.int32(VECS - 1))
            v0 = chunk_ref[pl.ds(mid * 16, 16)][0]
            big = v0 >= t
            return (jnp.where(active & ~big, mid + 1, lo_b),
                    jnp.where(active & big, mid, hi_b))
        b, _ = lax.fori_loop(0, 8, step, (jnp.int32(0), jnp.int32(VECS)))
        vb = chunk_ref[pl.ds(jnp.maximum(b - 1, 0) * 16, 16)]
        inblk = jnp.int32(0)
        for k in range(16):
            inblk = inblk + jnp.where(vb[k] < t, jnp.int32(1), jnp.int32(0))
        return jnp.where(b == 0, jnp.int32(0), (b - 1) * 16 + inblk)

    def per_bin(g, carry):
        p_prev, hs = carry
        p_next = count_less(g + 1)
        cnt = jnp.full((16,), p_next - p_prev, jnp.int32)
        gv = jnp.full((16,), g, jnp.int32)
        hs = tuple(
            jnp.where(lanes + (16 * k) == gv, hs[k] + cnt, hs[k])
            for k in range(4))
        return (p_next, hs)

    _, h = lax.fori_loop(
        lo, hi + 1, per_bin,
        (count_less(lo),
         tuple(jnp.zeros((16,), jnp.int32) for _ in range(4))))
    for k in range(4):
        hist_ref[pl.ds(16 * k, 16)] = h[k]
    pltpu.sync_copy(hist_ref, out_hbm.at[wid])


def _sc_bincount(idx_padded):
    mesh = plsc.VectorSubcoreMesh(core_axis_name="c", subcore_axis_name="s")
    kern = functools.partial(
        pl.kernel,
        mesh=mesh,
        out_type=jax.ShapeDtypeStruct((NW, N_GRAPH), jnp.int32),
        scratch_types=[
            pltpu.VMEM((CHUNK,), jnp.int32),
            pltpu.VMEM((N_GRAPH,), jnp.int32),
            pltpu.SemaphoreType.DMA,
        ],
    )(_sc_bincount_body)
    return kern(idx_padded)


def _tc_body(x_ref, idx_ref, pW1_ref, pb1_ref, pW2_ref, pb2_ref,
             spin_ref, emb0_ref, acc_ref):
    i = pl.program_id(0)
    nb = pl.num_programs(0)

    xb = x_ref[...]
    idx = idx_ref[0, 0, :]

    h = jnp.dot(xb, pW1_ref[...], preferred_element_type=jnp.float32)
    h = jnp.maximum(h + pb1_ref[...], 0.0)
    s = jnp.dot(h, pW2_ref[...], preferred_element_type=jnp.float32)
    spin_ref[...] = jnp.maximum(s + pb2_ref[...], 0.0)

    @pl.when(i == 0)
    def _init():
        acc_ref[...] = jnp.zeros_like(acc_ref)

    @pl.when(idx[0] == 0)
    def _emb0():
        m = (idx == 0).astype(jnp.float32)[:, None]
        acc_ref[...] += jnp.sum(xb * m, axis=0, keepdims=True)

    @pl.when(i == nb - 1)
    def _out():
        emb0_ref[...] = acc_ref[...]


def _value_head_body(cnt_ref, emb0_ref, vW1_ref, vb1_ref, vW2_ref, vb2_ref,
                     vW3_ref, vb3_ref, values_ref):
    n = jnp.sum(cnt_ref[...].astype(jnp.float32), axis=0)
    scale = jnp.sqrt(n).reshape(N_GRAPH, 1)
    z = emb0_ref[...] / scale
    h1 = jnp.dot(z, vW1_ref[...], preferred_element_type=jnp.float32)
    h1 = jnp.maximum(h1 + vb1_ref[...], 0.0)
    h2 = jnp.dot(h1, vW2_ref[...], preferred_element_type=jnp.float32)
    h2 = jnp.maximum(h2 + vb2_ref[...], 0.0)
    h3 = jnp.dot(h2, vW3_ref[...], preferred_element_type=jnp.float32)
    values_ref[...] = jnp.maximum(h3 + vb3_ref[...], 0.0)


@jax.jit
def _run(x, idx, pW1, pb1, pW2, pb2, vW1, vb1, vW2, vb2, vW3, vb3):
    n_nodes, d_feat = x.shape
    block = 10000
    if n_nodes % block != 0:
        block = 8
        while n_nodes % (block * 2) == 0 and block < 4096:
            block *= 2
    nb = n_nodes // block

    idx_padded = jnp.concatenate(
        [idx, jnp.full((NW * CHUNK - n_nodes,), N_GRAPH, jnp.int32)])
    partial_counts = _sc_bincount(idx_padded)

    idx3 = idx.reshape(nb, 1, block)
    spin, emb0 = pl.pallas_call(
        _tc_body,
        grid=(nb,),
        in_specs=[
            pl.BlockSpec((block, d_feat), lambda i: (i, 0)),
            pl.BlockSpec((1, 1, block), lambda i: (i, 0, 0)),
            pl.BlockSpec(pW1.shape, lambda i: (0, 0)),
            pl.BlockSpec((1, pb1.shape[0]), lambda i: (0, 0)),
            pl.BlockSpec(pW2.shape, lambda i: (0, 0)),
            pl.BlockSpec((1, pb2.shape[0]), lambda i: (0, 0)),
        ],
        out_specs=[
            pl.BlockSpec((block, 2), lambda i: (i, 0)),
            pl.BlockSpec((1, d_feat), lambda i: (0, 0)),
        ],
        out_shape=[
            jax.ShapeDtypeStruct((n_nodes, 2), jnp.float32),
            jax.ShapeDtypeStruct((1, d_feat), jnp.float32),
        ],
        scratch_shapes=[pltpu.VMEM((1, d_feat), jnp.float32)],
    )(x, idx3, pW1, pb1.reshape(1, -1), pW2, pb2.reshape(1, -1))

    values = pl.pallas_call(
        _value_head_body,
        out_shape=jax.ShapeDtypeStruct((N_GRAPH, 1), jnp.float32),
    )(partial_counts, emb0, vW1, vb1.reshape(1, -1),
      vW2, vb2.reshape(1, -1), vW3, vb3.reshape(1, -1))
    return spin, values[:, 0]


def kernel(x, node_graph_idx, pW1, pb1, pW2, pb2, vW1, vb1, vW2, vb2, vW3, vb3):
    idx = node_graph_idx.astype(jnp.int32)
    return _run(x, idx, pW1, pb1, pW2, pb2, vW1, vb1, vW2, vb2, vW3, vb3)


# speedup vs baseline: 1.2161x; 1.2161x over previous
import functools

import jax
import jax.numpy as jnp
from jax.experimental import pallas as pl
from jax.experimental.pallas import tpu as pltpu

N_GRAPH = 64


def _body(x_ref, idx_ref, pW1_ref, pb1_ref, pW2_ref, pb2_ref,
          vW1_ref, vb1_ref, vW2_ref, vb2_ref, vW3_ref, vb3_ref,
          spin_ref, values_ref, counts_ref, emb0_ref):
    i = pl.program_id(0)
    nb = pl.num_programs(0)

    xb = x_ref[...]

    idx = idx_ref[0, 0, :]
    bsz = idx.shape[0]
    gids = jax.lax.broadcasted_iota(jnp.int32, (N_GRAPH, bsz), 0)
    onehot = (idx[None, :] == gids).astype(jnp.float32)
    ones_col = jnp.ones((bsz, 1), jnp.float32)
    blk_counts = jnp.dot(onehot, ones_col,
                         preferred_element_type=jnp.float32)

    h = jnp.dot(xb, pW1_ref[...], preferred_element_type=jnp.float32)
    h = jnp.maximum(h + pb1_ref[...], 0.0)
    s = jnp.dot(h, pW2_ref[...], preferred_element_type=jnp.float32)
    spin_ref[...] = jnp.maximum(s + pb2_ref[...], 0.0)

    @pl.when(i == 0)
    def _init():
        counts_ref[...] = blk_counts
        emb0_ref[...] = jnp.zeros_like(emb0_ref)

    @pl.when(i != 0)
    def _acc():
        counts_ref[...] += blk_counts

    @pl.when(idx[0] == 0)
    def _emb0():
        m = (idx == 0).astype(jnp.float32)[:, None]
        emb0_ref[...] += jnp.sum(xb * m, axis=0, keepdims=True)

    @pl.when(i == nb - 1)
    def _final():
        scale = jnp.sqrt(counts_ref[...])
        z = emb0_ref[...] / scale
        h1 = jnp.dot(z, vW1_ref[...], preferred_element_type=jnp.float32)
        h1 = jnp.maximum(h1 + vb1_ref[...], 0.0)
        h2 = jnp.dot(h1, vW2_ref[...], preferred_element_type=jnp.float32)
        h2 = jnp.maximum(h2 + vb2_ref[...], 0.0)
        h3 = jnp.dot(h2, vW3_ref[...], preferred_element_type=jnp.float32)
        values_ref[...] = jnp.maximum(h3 + vb3_ref[...], 0.0)


@functools.partial(jax.jit, static_argnames=("interpret",))
def _run(x, idx, pW1, pb1, pW2, pb2, vW1, vb1, vW2, vb2, vW3, vb3,
         interpret=False):
    n_nodes, d_feat = x.shape
    block = 20000
    if n_nodes % block != 0:
        block = 8
        while n_nodes % (block * 2) == 0 and block < 4096:
            block *= 2
    nb = n_nodes // block

    idx3 = idx.reshape(nb, 1, block)

    in_specs = [
            pl.BlockSpec((block, d_feat), lambda i: (i, 0)),
            pl.BlockSpec((1, 1, block), lambda i: (i, 0, 0)),
            pl.BlockSpec(pW1.shape, lambda i: (0, 0)),
            pl.BlockSpec((1, pb1.shape[0]), lambda i: (0, 0)),
            pl.BlockSpec(pW2.shape, lambda i: (0, 0)),
            pl.BlockSpec((1, pb2.shape[0]), lambda i: (0, 0)),
            pl.BlockSpec(vW1.shape, lambda i: (0, 0)),
            pl.BlockSpec((1, vb1.shape[0]), lambda i: (0, 0)),
            pl.BlockSpec(vW2.shape, lambda i: (0, 0)),
            pl.BlockSpec((1, vb2.shape[0]), lambda i: (0, 0)),
            pl.BlockSpec(vW3.shape, lambda i: (0, 0)),
            pl.BlockSpec((1, vb3.shape[0]), lambda i: (0, 0)),
    ]
    out_specs = [
        pl.BlockSpec((block, 2), lambda i: (i, 0)),
        pl.BlockSpec((N_GRAPH, 1), lambda i: (0, 0)),
    ]

    spin, values = pl.pallas_call(
        _body,
        grid=(nb,),
        in_specs=in_specs,
        out_specs=out_specs,
        out_shape=[
            jax.ShapeDtypeStruct((n_nodes, 2), jnp.float32),
            jax.ShapeDtypeStruct((N_GRAPH, 1), jnp.float32),
        ],
        scratch_shapes=[
            pltpu.VMEM((N_GRAPH, 1), jnp.float32),
            pltpu.VMEM((1, d_feat), jnp.float32),
        ],
        interpret=interpret,
    )(x, idx3, pW1, pb1.reshape(1, -1), pW2, pb2.reshape(1, -1),
      vW1, vb1.reshape(1, -1), vW2, vb2.reshape(1, -1),
      vW3, vb3.reshape(1, -1))
    return spin, values[:, 0]


def kernel(x, node_graph_idx, pW1, pb1, pW2, pb2, vW1, vb1, vW2, vb2, vW3, vb3):
    idx = node_graph_idx.astype(jnp.int32)
    return _run(x, idx, pW1, pb1, pW2, pb2, vW1, vb1, vW2, vb2, vW3, vb3)


# speedup vs baseline: 1.2346x; 1.0152x over previous
import functools

import jax
import jax.numpy as jnp
from jax.experimental import pallas as pl
from jax.experimental.pallas import tpu as pltpu

N_GRAPH = 64


def _body(x_ref, idx_ref, pW1_ref, pb1_ref, pW2_ref, pb2_ref,
          vW1_ref, vb1_ref, vW2_ref, vb2_ref, vW3_ref, vb3_ref,
          spin_ref, values_ref, counts_ref, emb0_ref):
    i = pl.program_id(0)
    nb = pl.num_programs(0)

    xb = x_ref[...]

    idx = idx_ref[0, 0, :]
    bsz = idx.shape[0]
    gids = jax.lax.broadcasted_iota(jnp.int32, (N_GRAPH, bsz), 0)
    onehot = (idx[None, :] == gids).astype(jnp.float32)
    ones_col = jnp.ones((bsz, 1), jnp.float32)
    blk_counts = jnp.dot(onehot, ones_col,
                         preferred_element_type=jnp.float32)

    h = jnp.dot(xb, pW1_ref[...], preferred_element_type=jnp.float32)
    h = jnp.maximum(h + pb1_ref[...], 0.0)
    s = jnp.dot(h, pW2_ref[...], preferred_element_type=jnp.float32)
    spin_ref[...] = jnp.maximum(s + pb2_ref[...], 0.0)

    @pl.when(i == 0)
    def _init():
        counts_ref[...] = blk_counts
        emb0_ref[...] = jnp.zeros_like(emb0_ref)

    @pl.when(i != 0)
    def _acc():
        counts_ref[...] += blk_counts

    @pl.when(idx[0] == 0)
    def _emb0():
        m = (idx == 0).astype(jnp.float32)[:, None]
        emb0_ref[...] += jnp.sum(xb * m, axis=0, keepdims=True)

    @pl.when(i == nb - 1)
    def _final():
        scale = jnp.sqrt(counts_ref[...])
        z = emb0_ref[...] / scale
        h1 = jnp.dot(z, vW1_ref[...], preferred_element_type=jnp.float32)
        h1 = jnp.maximum(h1 + vb1_ref[...], 0.0)
        h2 = jnp.dot(h1, vW2_ref[...], preferred_element_type=jnp.float32)
        h2 = jnp.maximum(h2 + vb2_ref[...], 0.0)
        h3 = jnp.dot(h2, vW3_ref[...], preferred_element_type=jnp.float32)
        values_ref[...] = jnp.maximum(h3 + vb3_ref[...], 0.0)


@functools.partial(jax.jit, static_argnames=("interpret",))
def _run(x, idx, pW1, pb1, pW2, pb2, vW1, vb1, vW2, vb2, vW3, vb3,
         interpret=False):
    n_nodes, d_feat = x.shape
    block = 10000
    if n_nodes % block != 0:
        block = 8
        while n_nodes % (block * 2) == 0 and block < 4096:
            block *= 2
    nb = n_nodes // block

    idx3 = idx.reshape(nb, 1, block)

    in_specs = [
            pl.BlockSpec((block, d_feat), lambda i: (i, 0)),
            pl.BlockSpec((1, 1, block), lambda i: (i, 0, 0)),
            pl.BlockSpec(pW1.shape, lambda i: (0, 0)),
            pl.BlockSpec((1, pb1.shape[0]), lambda i: (0, 0)),
            pl.BlockSpec(pW2.shape, lambda i: (0, 0)),
            pl.BlockSpec((1, pb2.shape[0]), lambda i: (0, 0)),
            pl.BlockSpec(vW1.shape, lambda i: (0, 0)),
            pl.BlockSpec((1, vb1.shape[0]), lambda i: (0, 0)),
            pl.BlockSpec(vW2.shape, lambda i: (0, 0)),
            pl.BlockSpec((1, vb2.shape[0]), lambda i: (0, 0)),
            pl.BlockSpec(vW3.shape, lambda i: (0, 0)),
            pl.BlockSpec((1, vb3.shape[0]), lambda i: (0, 0)),
    ]
    out_specs = [
        pl.BlockSpec((block, 2), lambda i: (i, 0)),
        pl.BlockSpec((N_GRAPH, 1), lambda i: (0, 0)),
    ]

    spin, values = pl.pallas_call(
        _body,
        grid=(nb,),
        in_specs=in_specs,
        out_specs=out_specs,
        out_shape=[
            jax.ShapeDtypeStruct((n_nodes, 2), jnp.float32),
            jax.ShapeDtypeStruct((N_GRAPH, 1), jnp.float32),
        ],
        scratch_shapes=[
            pltpu.VMEM((N_GRAPH, 1), jnp.float32),
            pltpu.VMEM((1, d_feat), jnp.float32),
        ],
        interpret=interpret,
    )(x, idx3, pW1, pb1.reshape(1, -1), pW2, pb2.reshape(1, -1),
      vW1, vb1.reshape(1, -1), vW2, vb2.reshape(1, -1),
      vW3, vb3.reshape(1, -1))
    return spin, values[:, 0]


def kernel(x, node_graph_idx, pW1, pb1, pW2, pb2, vW1, vb1, vW2, vb2, vW3, vb3):
    idx = node_graph_idx.astype(jnp.int32)
    return _run(x, idx, pW1, pb1, pW2, pb2, vW1, vb1, vW2, vb2, vW3, vb3)
